# baseline (speedup 1.0000x reference)
import functools

import jax
import jax.numpy as jnp
from jax import lax
from jax.experimental import pallas as pl
from jax.experimental.pallas import tpu as pltpu
from jax.experimental.pallas import tpu_sc as plsc

_LW = 128
_NSLICE = 2
_TN = 4096


def _tc_prepare(table, w_param, b_param, w_out, b_out):
    v, h = table.shape
    d = w_out.shape[0]
    p = w_param.shape[1]

    def body(t_ref, wp_ref, bp_ref, wo_ref, bo_ref, tp_ref, wc_ref, be_ref):
        wo = wo_ref[...]
        w1 = wo[:, :h]
        w2 = wo[:, h:]
        dn_t = (((1,), (1,)), ((), ()))
        be_ref[...] = lax.dot_general(bp_ref[...], w2, dn_t,
                                      preferred_element_type=jnp.float32) + bo_ref[...]
        tp_ref[...] = lax.dot_general(t_ref[...], w1, dn_t,
                                      preferred_element_type=jnp.float32).astype(jnp.bfloat16)
        wc_ref[...] = lax.dot_general(wp_ref[...], w2,
                                      (((0,), (1,)), ((), ())),
                                      preferred_element_type=jnp.float32)

    return pl.pallas_call(
        body,
        out_shape=(
            jax.ShapeDtypeStruct((v, d), jnp.bfloat16),
            jax.ShapeDtypeStruct((p, d), jnp.float32),
            jax.ShapeDtypeStruct((1, d), jnp.float32),
        ),
    )(table, w_param, b_param, w_out, b_out)


def _sc_gather(ids3d, table_proj, s, nslice):
    blk_total, idxrows_per_w, lw = ids3d.shape
    v, d = table_proj.shape
    info = plsc.get_sparse_core_info()
    nw = info.num_cores * info.num_subcores
    assert blk_total == nslice * nw
    n = nw * idxrows_per_w * lw
    ch = 5
    nch = idxrows_per_w // ch
    rows_per_chunk = ch * lw
    rows_per_w = idxrows_per_w * lw
    assert nch * ch == idxrows_per_w

    mesh = plsc.VectorSubcoreMesh(core_axis_name="c", subcore_axis_name="s")

    @functools.partial(
        pl.kernel,
        out_type=jax.ShapeDtypeStruct((n, d), jnp.bfloat16),
        mesh=mesh,
        scratch_types=[
            pltpu.VMEM((idxrows_per_w, lw), jnp.int32),
            pltpu.VMEM((rows_per_chunk, d), jnp.bfloat16),
            pltpu.VMEM((rows_per_chunk, d), jnp.bfloat16),
            pltpu.SemaphoreType.DMA,
            pltpu.SemaphoreType.DMA,
        ],
        compiler_params=pltpu.CompilerParams(use_tc_tiling_on_sc=False),
    )
    def k(ids_hbm, table_hbm, out_hbm, idx_v, rows_a, rows_b, gsem, wsem):
        wid = lax.axis_index("s") * info.num_cores + lax.axis_index("c")
        row_base = wid * rows_per_w
        pltpu.sync_copy(ids_hbm.at[s * nw + wid], idx_v)

        bufs = (rows_a, rows_b)
        wb = [None, None]
        for c in range(nch):
            buf = bufs[c % 2]
            if wb[c % 2] is not None:
                wb[c % 2].wait()
            copies = [
                pltpu.async_copy(
                    table_hbm.at[idx_v.at[c * ch + j]],
                    buf.at[pl.ds(j * lw, lw)],
                    gsem,
                )
                for j in range(ch)
            ]
            for cp in copies:
                cp.wait()
            out_off = pl.multiple_of(row_base + c * rows_per_chunk, 8)
            wb[c % 2] = pltpu.async_copy(
                buf, out_hbm.at[pl.ds(out_off, rows_per_chunk)], wsem
            )
        for h in wb:
            if h is not None:
                h.wait()

    return k(ids3d, table_proj)


def _tc_combine_slice(gathered_s, params_2d, wc_t, b_eff, prev, s, n, tn=_TN):
    ns, d = gathered_s.shape
    p = wc_t.shape[0]
    nblk = ns // tn
    blk0 = s * nblk
    assert nblk * tn == ns

    def body(g_ref, pk_ref, wc_ref, be_ref, *o_refs):
        o_ref = o_refs[-1]
        pe = lax.dot_general(pk_ref[...], wc_ref[...], (((1,), (0,)), ((), ())),
                             preferred_element_type=jnp.float32)
        o_ref[...] = g_ref[...].astype(jnp.float32) + pe + be_ref[...]

    in_specs = [
        pl.BlockSpec((tn, d), lambda i: (i, 0)),
        pl.BlockSpec((tn, p), lambda i: (blk0 + i, 0)),
        pl.BlockSpec((p, d), lambda i: (0, 0)),
        pl.BlockSpec((1, d), lambda i: (0, 0)),
    ]
    args = [gathered_s, params_2d, wc_t, b_eff]
    aliases = {}
    if prev is not None:
        in_specs.append(pl.BlockSpec(memory_space=pl.ANY))
        args.append(prev)
        aliases = {4: 0}

    return pl.pallas_call(
        body,
        grid=(nblk,),
        in_specs=in_specs,
        out_specs=pl.BlockSpec((tn, d), lambda i: (blk0 + i, 0)),
        out_shape=jax.ShapeDtypeStruct((n, d), jnp.float32),
        input_output_aliases=aliases,
    )(*args)


def kernel(propensity_type_ids, propensity_params, type_table, W_param, b_param, W_out, b_out):
    b, r = propensity_type_ids.shape
    _, _, p = propensity_params.shape
    v, h = type_table.shape
    d = W_out.shape[0]
    n = b * r
    table_proj, wc_t, b_eff = _tc_prepare(
        type_table, W_param, b_param.reshape(1, h), W_out, b_out.reshape(1, d)
    )
    info = plsc.get_sparse_core_info()
    nw = info.num_cores * info.num_subcores
    ns = n // _NSLICE
    ids3d = propensity_type_ids.reshape(
        _NSLICE * nw, n // (_NSLICE * nw * _LW), _LW
    ).astype(jnp.int32)
    params_2d = propensity_params.reshape(n, p)

    gathered = [_sc_gather(ids3d, table_proj, s, _NSLICE) for s in range(_NSLICE)]
    out = None
    for s in range(_NSLICE):
        out = _tc_combine_slice(gathered[s], params_2d, wc_t, b_eff, out, s, n)
    return out.reshape(b, r, d)


# speedup vs baseline: 1.6256x; 1.6256x over previous
import functools

import jax
import jax.numpy as jnp
from jax import lax
from jax.experimental import pallas as pl
from jax.experimental.pallas import tpu as pltpu
from jax.experimental.pallas import tpu_sc as plsc

_LW = 128
_NSLICE = 2
_TN = 4096


def _tc_prepare(table, w_param, b_param, w_out, b_out):
    v, h = table.shape
    d = w_out.shape[0]
    p = w_param.shape[1]

    def body(t_ref, wp_ref, bp_ref, wo_ref, bo_ref, tp_ref, wc_ref, be_ref):
        wo = wo_ref[...]
        w1 = wo[:, :h]
        w2 = wo[:, h:]
        dn_t = (((1,), (1,)), ((), ()))
        be_ref[...] = lax.dot_general(bp_ref[...], w2, dn_t,
                                      preferred_element_type=jnp.float32) + bo_ref[...]
        tp_ref[...] = lax.dot_general(t_ref[...], w1, dn_t,
                                      preferred_element_type=jnp.float32)
        wc_ref[...] = lax.dot_general(wp_ref[...], w2,
                                      (((0,), (1,)), ((), ())),
                                      preferred_element_type=jnp.float32)

    return pl.pallas_call(
        body,
        out_shape=(
            jax.ShapeDtypeStruct((v, d), jnp.float32),
            jax.ShapeDtypeStruct((p, d), jnp.float32),
            jax.ShapeDtypeStruct((1, d), jnp.float32),
        ),
    )(table, w_param, b_param, w_out, b_out)


def _sc_gather(ids3d, table_proj, s, nslice):
    blk_total, idxrows_per_w, lw = ids3d.shape
    v, d = table_proj.shape
    info = plsc.get_sparse_core_info()
    nw = info.num_cores * info.num_subcores
    assert blk_total == nslice * nw
    n = nw * idxrows_per_w * lw
    ch = 5
    nch = idxrows_per_w // ch
    rows_per_chunk = ch * lw
    rows_per_w = idxrows_per_w * lw
    assert nch * ch == idxrows_per_w

    mesh = plsc.VectorSubcoreMesh(core_axis_name="c", subcore_axis_name="s")

    @functools.partial(
        pl.kernel,
        out_type=jax.ShapeDtypeStruct((n, d), jnp.float32),
        mesh=mesh,
        scratch_types=[
            pltpu.VMEM((idxrows_per_w, lw), jnp.int32),
            pltpu.VMEM((rows_per_chunk, d), jnp.float32),
            pltpu.SemaphoreType.DMA,
        ],
    )
    def k(ids_hbm, table_hbm, out_hbm, idx_v, rows_v, sem):
        wid = lax.axis_index("s") * info.num_cores + lax.axis_index("c")
        row_base = wid * rows_per_w
        pltpu.sync_copy(ids_hbm.at[s * nw + wid], idx_v)

        def body(c, carry):
            copies = [
                pltpu.async_copy(
                    table_hbm.at[idx_v.at[c * ch + j]],
                    rows_v.at[pl.ds(j * lw, lw)],
                    sem,
                )
                for j in range(ch)
            ]
            for cp in copies:
                cp.wait()
            out_off = pl.multiple_of(row_base + c * rows_per_chunk, 8)
            pltpu.sync_copy(rows_v, out_hbm.at[pl.ds(out_off, rows_per_chunk)])
            return carry

        lax.fori_loop(0, nch, body, 0)

    return k(ids3d, table_proj)


def _tc_combine_slice(gathered_s, params_2d, wc_t, b_eff, prev, s, n, tn=_TN):
    ns, d = gathered_s.shape
    p = wc_t.shape[0]
    nblk = ns // tn
    blk0 = s * nblk
    assert nblk * tn == ns

    def body(g_ref, pk_ref, wc_ref, be_ref, *o_refs):
        o_ref = o_refs[-1]
        pe = lax.dot_general(pk_ref[...], wc_ref[...], (((1,), (0,)), ((), ())),
                             preferred_element_type=jnp.float32)
        o_ref[...] = g_ref[...] + pe + be_ref[...]

    in_specs = [
        pl.BlockSpec((tn, d), lambda i: (i, 0)),
        pl.BlockSpec((tn, p), lambda i: (blk0 + i, 0)),
        pl.BlockSpec((p, d), lambda i: (0, 0)),
        pl.BlockSpec((1, d), lambda i: (0, 0)),
    ]
    args = [gathered_s, params_2d, wc_t, b_eff]
    aliases = {}
    if prev is not None:
        in_specs.append(pl.BlockSpec(memory_space=pl.ANY))
        args.append(prev)
        aliases = {4: 0}

    return pl.pallas_call(
        body,
        grid=(nblk,),
        in_specs=in_specs,
        out_specs=pl.BlockSpec((tn, d), lambda i: (blk0 + i, 0)),
        out_shape=jax.ShapeDtypeStruct((n, d), jnp.float32),
        input_output_aliases=aliases,
    )(*args)


def kernel(propensity_type_ids, propensity_params, type_table, W_param, b_param, W_out, b_out):
    b, r = propensity_type_ids.shape
    _, _, p = propensity_params.shape
    v, h = type_table.shape
    d = W_out.shape[0]
    n = b * r
    table_proj, wc_t, b_eff = _tc_prepare(
        type_table, W_param, b_param.reshape(1, h), W_out, b_out.reshape(1, d)
    )
    info = plsc.get_sparse_core_info()
    nw = info.num_cores * info.num_subcores
    ns = n // _NSLICE
    ids3d = propensity_type_ids.reshape(
        _NSLICE * nw, n // (_NSLICE * nw * _LW), _LW
    ).astype(jnp.int32)
    params_2d = propensity_params.reshape(n, p)

    gathered = [_sc_gather(ids3d, table_proj, s, _NSLICE) for s in range(_NSLICE)]
    out = None
    for s in range(_NSLICE):
        out = _tc_combine_slice(gathered[s], params_2d, wc_t, b_eff, out, s, n)
    return out.reshape(b, r, d)


# speedup vs baseline: 1.6511x; 1.0157x over previous
import functools

import jax
import jax.numpy as jnp
from jax import lax
from jax.experimental import pallas as pl
from jax.experimental.pallas import tpu as pltpu
from jax.experimental.pallas import tpu_sc as plsc

_LW = 128
_NSLICE = 2
_TN = 10240


def _tc_prepare(table, w_param, b_param, w_out, b_out):
    v, h = table.shape
    d = w_out.shape[0]
    p = w_param.shape[1]

    def body(t_ref, wp_ref, bp_ref, wo_ref, bo_ref, tp_ref, wc_ref, be_ref):
        wo = wo_ref[...]
        w1 = wo[:, :h]
        w2 = wo[:, h:]
        dn_t = (((1,), (1,)), ((), ()))
        be_ref[...] = lax.dot_general(bp_ref[...], w2, dn_t,
                                      preferred_element_type=jnp.float32) + bo_ref[...]
        tp_ref[...] = lax.dot_general(t_ref[...], w1, dn_t,
                                      preferred_element_type=jnp.float32)
        wc_ref[...] = lax.dot_general(wp_ref[...], w2,
                                      (((0,), (1,)), ((), ())),
                                      preferred_element_type=jnp.float32)

    return pl.pallas_call(
        body,
        out_shape=(
            jax.ShapeDtypeStruct((v, d), jnp.float32),
            jax.ShapeDtypeStruct((p, d), jnp.float32),
            jax.ShapeDtypeStruct((1, d), jnp.float32),
        ),
    )(table, w_param, b_param, w_out, b_out)


def _sc_gather(ids3d, table_proj, s, nslice):
    blk_total, idxrows_per_w, lw = ids3d.shape
    v, d = table_proj.shape
    info = plsc.get_sparse_core_info()
    nw = info.num_cores * info.num_subcores
    assert blk_total == nslice * nw
    n = nw * idxrows_per_w * lw
    ch = 5
    nch = idxrows_per_w // ch
    rows_per_chunk = ch * lw
    rows_per_w = idxrows_per_w * lw
    assert nch * ch == idxrows_per_w

    mesh = plsc.VectorSubcoreMesh(core_axis_name="c", subcore_axis_name="s")

    @functools.partial(
        pl.kernel,
        out_type=jax.ShapeDtypeStruct((n, d), jnp.float32),
        mesh=mesh,
        scratch_types=[
            pltpu.VMEM((idxrows_per_w, lw), jnp.int32),
            pltpu.VMEM((rows_per_chunk, d), jnp.float32),
            pltpu.SemaphoreType.DMA,
        ],
    )
    def k(ids_hbm, table_hbm, out_hbm, idx_v, rows_v, sem):
        wid = lax.axis_index("s") * info.num_cores + lax.axis_index("c")
        row_base = wid * rows_per_w
        pltpu.sync_copy(ids_hbm.at[s * nw + wid], idx_v)

        def body(c, carry):
            copies = [
                pltpu.async_copy(
                    table_hbm.at[idx_v.at[c * ch + j]],
                    rows_v.at[pl.ds(j * lw, lw)],
                    sem,
                )
                for j in range(ch)
            ]
            for cp in copies:
                cp.wait()
            out_off = pl.multiple_of(row_base + c * rows_per_chunk, 8)
            pltpu.sync_copy(rows_v, out_hbm.at[pl.ds(out_off, rows_per_chunk)])
            return carry

        lax.fori_loop(0, nch, body, 0)

    return k(ids3d, table_proj)


def _tc_combine_slice(gathered_s, params_2d, wc_t, b_eff, prev, s, n, tn=_TN):
    ns, d = gathered_s.shape
    p = wc_t.shape[0]
    nblk = ns // tn
    blk0 = s * nblk
    assert nblk * tn == ns

    def body(g_ref, pk_ref, wc_ref, be_ref, *o_refs):
        o_ref = o_refs[-1]
        pe = lax.dot_general(pk_ref[...], wc_ref[...], (((1,), (0,)), ((), ())),
                             preferred_element_type=jnp.float32)
        o_ref[...] = g_ref[...] + pe + be_ref[...]

    in_specs = [
        pl.BlockSpec((tn, d), lambda i: (i, 0)),
        pl.BlockSpec((tn, p), lambda i: (blk0 + i, 0)),
        pl.BlockSpec((p, d), lambda i: (0, 0)),
        pl.BlockSpec((1, d), lambda i: (0, 0)),
    ]
    args = [gathered_s, params_2d, wc_t, b_eff]
    aliases = {}
    if prev is not None:
        in_specs.append(pl.BlockSpec(memory_space=pl.ANY))
        args.append(prev)
        aliases = {4: 0}

    return pl.pallas_call(
        body,
        grid=(nblk,),
        in_specs=in_specs,
        out_specs=pl.BlockSpec((tn, d), lambda i: (blk0 + i, 0)),
        out_shape=jax.ShapeDtypeStruct((n, d), jnp.float32),
        input_output_aliases=aliases,
    )(*args)


def kernel(propensity_type_ids, propensity_params, type_table, W_param, b_param, W_out, b_out):
    b, r = propensity_type_ids.shape
    _, _, p = propensity_params.shape
    v, h = type_table.shape
    d = W_out.shape[0]
    n = b * r
    table_proj, wc_t, b_eff = _tc_prepare(
        type_table, W_param, b_param.reshape(1, h), W_out, b_out.reshape(1, d)
    )
    info = plsc.get_sparse_core_info()
    nw = info.num_cores * info.num_subcores
    ns = n // _NSLICE
    ids3d = propensity_type_ids.reshape(
        _NSLICE * nw, n // (_NSLICE * nw * _LW), _LW
    ).astype(jnp.int32)
    params_2d = propensity_params.reshape(n, p)

    gathered = [_sc_gather(ids3d, table_proj, s, _NSLICE) for s in range(_NSLICE)]
    out = None
    for s in range(_NSLICE):
        out = _tc_combine_slice(gathered[s], params_2d, wc_t, b_eff, out, s, n)
    return out.reshape(b, r, d)
